# baseline (speedup 1.0000x reference)
import functools

import jax
import jax.numpy as jnp
from jax import lax
from jax.experimental import pallas as pl
from jax.experimental.pallas import tpu as pltpu
from jax.experimental.pallas import tpu_sc as plsc

N, D, E, K, F = 2048, 1024, 8, 2, 2048
B = 512
G = (N * K) // B + E
S = G * B
NEG = -3.0e38


def _cumsum_lanes(a):
    n = a.shape[1]
    sh = 1
    while sh < n:
        z = jnp.zeros((a.shape[0], sh), a.dtype)
        a = a + jnp.concatenate([z, a[:, :-sh]], axis=1)
        sh *= 2
    return a


def _router_body(x_ref, wp_ref, bp_ref, pos1_ref, pos2_ref, w1_ref, w2_ref,
                 blk_ref, nblk_ref):
    x = x_ref[...]
    wp = wp_ref[...]
    lg = lax.dot_general(wp, x, (((0,), (1,)), ((), ())),
                         preferred_element_type=jnp.float32)
    lg = lg + bp_ref[...]
    eidx = lax.broadcasted_iota(jnp.int32, (E, N), 0)
    v1 = jnp.max(lg, axis=0, keepdims=True)
    i1 = jnp.min(jnp.where(lg == v1, eidx, E), axis=0, keepdims=True)
    masked = jnp.where(eidx == i1, NEG, lg)
    v2 = jnp.max(masked, axis=0, keepdims=True)
    i2 = jnp.min(jnp.where(masked == v2, eidx, E), axis=0, keepdims=True)
    s = jnp.exp(v2 - v1)
    w1 = 1.0 / (1.0 + s)
    w2 = s / (1.0 + s)
    oh1 = (eidx == i1).astype(jnp.int32)
    oh2 = (eidx == i2).astype(jnp.int32)
    c1 = _cumsum_lanes(oh1)
    c2 = _cumsum_lanes(oh2)
    t1 = c1[:, -1:]
    t2 = c2[:, -1:]
    counts = t1 + t2
    pc = ((counts + (B - 1)) // B) * B
    inc = pc
    sh = 1
    while sh < E:
        z = jnp.zeros((sh, 1), jnp.int32)
        inc = inc + jnp.concatenate([z, inc[:-sh, :]], axis=0)
        sh *= 2
    offs = inc - pc
    total = inc[-1:, :]
    pos1 = jnp.sum(oh1 * (offs + c1 - 1), axis=0)
    pos2 = jnp.sum(oh2 * (offs + t1 + c2 - 1), axis=0)
    pos1_ref[...] = pos1
    pos2_ref[...] = pos2
    w1_ref[...] = jnp.broadcast_to(w1.reshape(N, 1), (N, 16))
    w2_ref[...] = jnp.broadcast_to(w2.reshape(N, 1), (N, 16))
    bb = lax.broadcasted_iota(jnp.int32, (1, G), 1) * B
    blk_ref[...] = jnp.sum((bb >= offs).astype(jnp.int32), axis=0) - 1
    nblk_ref[...] = (total // B).reshape(1)


def _router(x, Wp, bp, interpret=False):
    return pl.pallas_call(
        _router_body,
        out_shape=(
            jax.ShapeDtypeStruct((N,), jnp.int32),
            jax.ShapeDtypeStruct((N,), jnp.int32),
            jax.ShapeDtypeStruct((N, 16), jnp.float32),
            jax.ShapeDtypeStruct((N, 16), jnp.float32),
            jax.ShapeDtypeStruct((G,), jnp.int32),
            jax.ShapeDtypeStruct((1,), jnp.int32),
        ),
        interpret=interpret,
    )(x, Wp, bp.reshape(E, 1))


def _ffn_body(blk_ref, nblk_ref, xs_ref, w1_ref, b1_ref, w2_ref, b2_ref,
              ys_ref):
    g = pl.program_id(0)

    @pl.when(g < nblk_ref[0])
    def _():
        e = blk_ref[g]
        h = jnp.maximum(
            jnp.dot(xs_ref[...].astype(jnp.bfloat16),
                    w1_ref[0].astype(jnp.bfloat16),
                    preferred_element_type=jnp.float32)
            + b1_ref[pl.ds(e, 1)], 0.0)
        ys_ref[...] = jnp.dot(h.astype(jnp.bfloat16),
                              w2_ref[0].astype(jnp.bfloat16),
                              preferred_element_type=jnp.float32) \
            + b2_ref[pl.ds(e, 1)]


def _ffn(xs, W1, b1, W2, b2, blk_e, nblk, interpret=False):
    def xs_map(g, be, nb):
        return (jnp.where(g < nb[0], g, nb[0]), 0)

    def ys_map(g, be, nb):
        return (jnp.where(g < nb[0], g, G - 1), 0)

    grid_spec = pltpu.PrefetchScalarGridSpec(
        num_scalar_prefetch=2,
        grid=(G,),
        in_specs=[
            pl.BlockSpec((B, D), xs_map),
            pl.BlockSpec((1, D, F), lambda g, be, nb: (be[g], 0, 0)),
            pl.BlockSpec((E, F), lambda g, be, nb: (0, 0)),
            pl.BlockSpec((1, F, D), lambda g, be, nb: (be[g], 0, 0)),
            pl.BlockSpec((E, D), lambda g, be, nb: (0, 0)),
        ],
        out_specs=pl.BlockSpec((B, D), ys_map),
    )
    return pl.pallas_call(
        _ffn_body,
        grid_spec=grid_spec,
        out_shape=jax.ShapeDtypeStruct((S, D), jnp.float32),
        compiler_params=pltpu.CompilerParams(
            dimension_semantics=("arbitrary",)),
        interpret=interpret,
    )(blk_e, nblk, xs, W1, b1, W2, b2)


_MESH = plsc.VectorSubcoreMesh(core_axis_name="c", subcore_axis_name="s")
_NW = 32
_TPW = N // _NW
_CH = _TPW // 16


def _dispatch_sc(x, pos1, pos2):
    H = _TPW // 2

    @functools.partial(
        pl.kernel, mesh=_MESH,
        out_type=jax.ShapeDtypeStruct((S, D), jnp.float32),
        scratch_types=[
            pltpu.VMEM((H,), jnp.int32),
            pltpu.VMEM((H,), jnp.int32),
            pltpu.VMEM((H,), jnp.int32),
            pltpu.VMEM((H,), jnp.int32),
            pltpu.VMEM((H, D), jnp.float32),
            pltpu.VMEM((H, D), jnp.float32),
            pltpu.SemaphoreType.DMA,
            pltpu.SemaphoreType.DMA,
        ],
    )
    def body(x_hbm, p1_hbm, p2_hbm, xs_hbm, idx1a, idx1b, idx2a, idx2b,
             rows_a, rows_b, isem, sem):
        wid = lax.axis_index("s") * 2 + lax.axis_index("c")
        base = wid * _TPW
        c1 = pltpu.async_copy(p1_hbm.at[pl.ds(base, H)], idx1a, isem)
        c2 = pltpu.async_copy(p2_hbm.at[pl.ds(base, H)], idx2a, isem)
        c3 = pltpu.async_copy(p1_hbm.at[pl.ds(base + H, H)], idx1b, isem)
        c4 = pltpu.async_copy(p2_hbm.at[pl.ds(base + H, H)], idx2b, isem)
        cra = pltpu.async_copy(x_hbm.at[pl.ds(base, H)], rows_a, isem)
        crb = pltpu.async_copy(x_hbm.at[pl.ds(base + H, H)], rows_b, isem)
        c1.wait(); c2.wait(); cra.wait()
        s1 = pltpu.async_copy(rows_a, xs_hbm.at[idx1a], sem)
        s2 = pltpu.async_copy(rows_a, xs_hbm.at[idx2a], sem)
        c3.wait(); c4.wait(); crb.wait()
        s3 = pltpu.async_copy(rows_b, xs_hbm.at[idx1b], sem)
        s4 = pltpu.async_copy(rows_b, xs_hbm.at[idx2b], sem)
        s1.wait(); s2.wait(); s3.wait(); s4.wait()

    return body(x, pos1, pos2)


def _combine_sc(ys, pos1, pos2, w1, w2):
    H = _TPW // 2

    @functools.partial(
        pl.kernel, mesh=_MESH,
        out_type=jax.ShapeDtypeStruct((N, D), jnp.float32),
        scratch_types=[
            pltpu.VMEM((_TPW,), jnp.int32),
            pltpu.VMEM((_TPW,), jnp.int32),
            pltpu.VMEM((_TPW, 16), jnp.float32),
            pltpu.VMEM((_TPW, 16), jnp.float32),
            pltpu.VMEM((H, D), jnp.float32),
            pltpu.VMEM((H, D), jnp.float32),
            pltpu.VMEM((H, D), jnp.float32),
            pltpu.SemaphoreType.DMA,
            pltpu.SemaphoreType.DMA,
        ],
    )
    def body(ys_hbm, p1_hbm, p2_hbm, w1_hbm, w2_hbm, out_hbm, idx1_v, idx2_v,
             wv1, wv2, bufa, bufb, bufc, sem, osem):
        wid = lax.axis_index("s") * 2 + lax.axis_index("c")
        base = wid * _TPW
        pltpu.sync_copy(p1_hbm.at[pl.ds(base, _TPW)], idx1_v)
        pltpu.sync_copy(p2_hbm.at[pl.ds(base, _TPW)], idx2_v)
        pltpu.sync_copy(w1_hbm.at[pl.ds(base, _TPW)], wv1)
        pltpu.sync_copy(w2_hbm.at[pl.ds(base, _TPW)], wv2)
        g1a = pltpu.async_copy(ys_hbm.at[idx1_v.at[pl.ds(0, H)]], bufa, sem)
        g2a = pltpu.async_copy(ys_hbm.at[idx2_v.at[pl.ds(0, H)]], bufb, sem)
        g1b = pltpu.async_copy(ys_hbm.at[idx1_v.at[pl.ds(H, H)]], bufc, sem)
        g1a.wait(); g2a.wait()

        def add_rows(dst, src, woff):
            def add_row(i, _):
                bw1 = wv1[woff + i, :]
                bw2 = wv2[woff + i, :]

                def add_chunk(j, _):
                    sl = pl.ds(j * 16, 16)
                    dst[i, sl] = dst[i, sl] * bw1 + src[i, sl] * bw2
                    return 0
                return lax.fori_loop(0, D // 16, add_chunk, 0, unroll=4)
            lax.fori_loop(0, H, add_row, 0)

        add_rows(bufa, bufb, 0)
        sa = pltpu.async_copy(bufa, out_hbm.at[pl.ds(base, H)], osem)
        g2b = pltpu.async_copy(ys_hbm.at[idx2_v.at[pl.ds(H, H)]], bufb, sem)
        g1b.wait(); g2b.wait()
        add_rows(bufc, bufb, H)
        sc = pltpu.async_copy(bufc, out_hbm.at[pl.ds(base + H, H)], osem)
        sa.wait(); sc.wait()

    return body(ys, pos1, pos2, w1, w2)


def kernel(x, Wp, bp, W1, b1, W2, b2):
    pos1, pos2, w1, w2, blk_e, nblk = _router(x, Wp, bp)
    xs = _dispatch_sc(x, pos1, pos2)
    ys = _ffn(xs, W1, b1, W2, b2, blk_e, nblk)
    return _combine_sc(ys, pos1, pos2, w1, w2)


# speedup vs baseline: 1.0333x; 1.0333x over previous
import functools

import jax
import jax.numpy as jnp
from jax import lax
from jax.experimental import pallas as pl
from jax.experimental.pallas import tpu as pltpu
from jax.experimental.pallas import tpu_sc as plsc

N, D, E, K, F = 2048, 1024, 8, 2, 2048
B = 512
G = (N * K) // B + E
S = G * B
NEG = -3.0e38


def _cumsum_lanes(a):
    n = a.shape[1]
    sh = 1
    while sh < n:
        z = jnp.zeros((a.shape[0], sh), a.dtype)
        a = a + jnp.concatenate([z, a[:, :-sh]], axis=1)
        sh *= 2
    return a


def _router_body(x_ref, wp_ref, bp_ref, pos1_ref, pos2_ref, w1_ref, w2_ref,
                 blk_ref, nblk_ref):
    x = x_ref[...]
    wp = wp_ref[...]
    lg = lax.dot_general(wp, x, (((0,), (1,)), ((), ())),
                         preferred_element_type=jnp.float32)
    lg = lg + bp_ref[...]
    eidx = lax.broadcasted_iota(jnp.int32, (E, N), 0)
    v1 = jnp.max(lg, axis=0, keepdims=True)
    i1 = jnp.min(jnp.where(lg == v1, eidx, E), axis=0, keepdims=True)
    masked = jnp.where(eidx == i1, NEG, lg)
    v2 = jnp.max(masked, axis=0, keepdims=True)
    i2 = jnp.min(jnp.where(masked == v2, eidx, E), axis=0, keepdims=True)
    s = jnp.exp(v2 - v1)
    w1 = 1.0 / (1.0 + s)
    w2 = s / (1.0 + s)
    oh1 = (eidx == i1).astype(jnp.int32)
    oh2 = (eidx == i2).astype(jnp.int32)
    c1 = _cumsum_lanes(oh1)
    c2 = _cumsum_lanes(oh2)
    t1 = c1[:, -1:]
    t2 = c2[:, -1:]
    counts = t1 + t2
    pc = ((counts + (B - 1)) // B) * B
    inc = pc
    sh = 1
    while sh < E:
        z = jnp.zeros((sh, 1), jnp.int32)
        inc = inc + jnp.concatenate([z, inc[:-sh, :]], axis=0)
        sh *= 2
    offs = inc - pc
    total = inc[-1:, :]
    pos1 = jnp.sum(oh1 * (offs + c1 - 1), axis=0)
    pos2 = jnp.sum(oh2 * (offs + t1 + c2 - 1), axis=0)
    pos1_ref[...] = pos1
    pos2_ref[...] = pos2
    w1_ref[...] = jnp.broadcast_to(w1.reshape(N, 1), (N, 16))
    w2_ref[...] = jnp.broadcast_to(w2.reshape(N, 1), (N, 16))
    bb = lax.broadcasted_iota(jnp.int32, (1, G), 1) * B
    blk_ref[...] = jnp.sum((bb >= offs).astype(jnp.int32), axis=0) - 1
    nblk_ref[...] = (total // B).reshape(1)


def _router(x, Wp, bp, interpret=False):
    return pl.pallas_call(
        _router_body,
        out_shape=(
            jax.ShapeDtypeStruct((N,), jnp.int32),
            jax.ShapeDtypeStruct((N,), jnp.int32),
            jax.ShapeDtypeStruct((N, 16), jnp.float32),
            jax.ShapeDtypeStruct((N, 16), jnp.float32),
            jax.ShapeDtypeStruct((G,), jnp.int32),
            jax.ShapeDtypeStruct((1,), jnp.int32),
        ),
        interpret=interpret,
    )(x, Wp, bp.reshape(E, 1))


def _ffn_body(blk_ref, nblk_ref, xs_ref, w1_ref, b1_ref, w2_ref, b2_ref,
              ys_ref):
    g = pl.program_id(0)

    @pl.when(g < nblk_ref[0])
    def _():
        e = blk_ref[g]
        h = jnp.maximum(
            jnp.dot(xs_ref[...], w1_ref[0], preferred_element_type=jnp.float32)
            + b1_ref[pl.ds(e, 1)], 0.0)
        ys_ref[...] = jnp.dot(h, w2_ref[0],
                              preferred_element_type=jnp.float32) \
            + b2_ref[pl.ds(e, 1)]


def _ffn(xs, W1, b1, W2, b2, blk_e, nblk, interpret=False):
    def xs_map(g, be, nb):
        return (jnp.where(g < nb[0], g, nb[0]), 0)

    def ys_map(g, be, nb):
        return (jnp.where(g < nb[0], g, G - 1), 0)

    grid_spec = pltpu.PrefetchScalarGridSpec(
        num_scalar_prefetch=2,
        grid=(G,),
        in_specs=[
            pl.BlockSpec((B, D), xs_map),
            pl.BlockSpec((1, D, F), lambda g, be, nb: (be[g], 0, 0)),
            pl.BlockSpec((E, F), lambda g, be, nb: (0, 0)),
            pl.BlockSpec((1, F, D), lambda g, be, nb: (be[g], 0, 0)),
            pl.BlockSpec((E, D), lambda g, be, nb: (0, 0)),
        ],
        out_specs=pl.BlockSpec((B, D), ys_map),
    )
    return pl.pallas_call(
        _ffn_body,
        grid_spec=grid_spec,
        out_shape=jax.ShapeDtypeStruct((S, D), jnp.float32),
        compiler_params=pltpu.CompilerParams(
            dimension_semantics=("arbitrary",)),
        interpret=interpret,
    )(blk_e, nblk, xs, W1, b1, W2, b2)


_MESH = plsc.VectorSubcoreMesh(core_axis_name="c", subcore_axis_name="s")
_NW = 32
_TPW = N // _NW
_CH = _TPW // 16


def _dispatch_sc(x, pos1, pos2):
    H = _TPW // 2

    @functools.partial(
        pl.kernel, mesh=_MESH,
        out_type=jax.ShapeDtypeStruct((S, D), jnp.float32),
        scratch_types=[
            pltpu.VMEM((H,), jnp.int32),
            pltpu.VMEM((H,), jnp.int32),
            pltpu.VMEM((H,), jnp.int32),
            pltpu.VMEM((H,), jnp.int32),
            pltpu.VMEM((H, D), jnp.float32),
            pltpu.VMEM((H, D), jnp.float32),
            pltpu.SemaphoreType.DMA,
            pltpu.SemaphoreType.DMA,
        ],
    )
    def body(x_hbm, p1_hbm, p2_hbm, xs_hbm, idx1a, idx1b, idx2a, idx2b,
             rows_a, rows_b, isem, sem):
        wid = lax.axis_index("s") * 2 + lax.axis_index("c")
        base = wid * _TPW
        c1 = pltpu.async_copy(p1_hbm.at[pl.ds(base, H)], idx1a, isem)
        c2 = pltpu.async_copy(p2_hbm.at[pl.ds(base, H)], idx2a, isem)
        c3 = pltpu.async_copy(p1_hbm.at[pl.ds(base + H, H)], idx1b, isem)
        c4 = pltpu.async_copy(p2_hbm.at[pl.ds(base + H, H)], idx2b, isem)
        cra = pltpu.async_copy(x_hbm.at[pl.ds(base, H)], rows_a, isem)
        crb = pltpu.async_copy(x_hbm.at[pl.ds(base + H, H)], rows_b, isem)
        c1.wait(); c2.wait(); cra.wait()
        s1 = pltpu.async_copy(rows_a, xs_hbm.at[idx1a], sem)
        s2 = pltpu.async_copy(rows_a, xs_hbm.at[idx2a], sem)
        c3.wait(); c4.wait(); crb.wait()
        s3 = pltpu.async_copy(rows_b, xs_hbm.at[idx1b], sem)
        s4 = pltpu.async_copy(rows_b, xs_hbm.at[idx2b], sem)
        s1.wait(); s2.wait(); s3.wait(); s4.wait()

    return body(x, pos1, pos2)


def _combine_sc(ys, pos1, pos2, w1, w2):
    Q = 16
    NQ = _TPW // Q

    @functools.partial(
        pl.kernel, mesh=_MESH,
        out_type=jax.ShapeDtypeStruct((N, D), jnp.float32),
        scratch_types=[
            pltpu.VMEM((_TPW,), jnp.int32),
            pltpu.VMEM((_TPW,), jnp.int32),
            pltpu.VMEM((_TPW, 16), jnp.float32),
            pltpu.VMEM((_TPW, 16), jnp.float32),
            pltpu.VMEM((6, Q, D), jnp.float32),
            pltpu.SemaphoreType.DMA,
            pltpu.SemaphoreType.DMA,
        ],
    )
    def body(ys_hbm, p1_hbm, p2_hbm, w1_hbm, w2_hbm, out_hbm, idx1_v, idx2_v,
             wv1, wv2, bufs, sem, osem):
        wid = lax.axis_index("s") * 2 + lax.axis_index("c")
        base = wid * _TPW
        pltpu.sync_copy(p1_hbm.at[pl.ds(base, _TPW)], idx1_v)
        pltpu.sync_copy(p2_hbm.at[pl.ds(base, _TPW)], idx2_v)
        pltpu.sync_copy(w1_hbm.at[pl.ds(base, _TPW)], wv1)
        pltpu.sync_copy(w2_hbm.at[pl.ds(base, _TPW)], wv2)

        def gathers(q):
            p = q % 3
            ga = pltpu.async_copy(ys_hbm.at[idx1_v.at[pl.ds(q * Q, Q)]],
                                  bufs.at[2 * p], sem)
            gb = pltpu.async_copy(ys_hbm.at[idx2_v.at[pl.ds(q * Q, Q)]],
                                  bufs.at[2 * p + 1], sem)
            return ga, gb

        def store(q):
            p = q % 3
            return pltpu.async_copy(bufs.at[2 * p],
                                    out_hbm.at[pl.ds(base + q * Q, Q)], osem)

        pend = [gathers(0), gathers(1), gathers(2)]
        stores = []
        for q in range(NQ):
            ga, gb = pend[q]
            ga.wait(); gb.wait()
            p = q % 3
            dst = bufs.at[2 * p]
            src = bufs.at[2 * p + 1]

            def add_row(i, _, dst=dst, src=src, woff=q * Q):
                bw1 = wv1[woff + i, :]
                bw2 = wv2[woff + i, :]

                def add_chunk(j, _):
                    sl = pl.ds(j * 16, 16)
                    dst[i, sl] = dst[i, sl] * bw1 + src[i, sl] * bw2
                    return 0
                return lax.fori_loop(0, D // 16, add_chunk, 0, unroll=4)

            lax.fori_loop(0, Q, add_row, 0)
            stores.append(store(q))
            if q + 3 < NQ:
                stores[q].wait()
                pend.append(gathers(q + 3))
        for q in range(max(0, NQ - 3), NQ):
            stores[q].wait()

    return body(ys, pos1, pos2, w1, w2)


def kernel(x, Wp, bp, W1, b1, W2, b2):
    pos1, pos2, w1, w2, blk_e, nblk = _router(x, Wp, bp)
    xs = _dispatch_sc(x, pos1, pos2)
    ys = _ffn(xs, W1, b1, W2, b2, blk_e, nblk)
    return _combine_sc(ys, pos1, pos2, w1, w2)
